# baseline (speedup 1.0000x reference)
import functools

import jax
import jax.numpy as jnp
from jax import lax
from jax.experimental import pallas as pl
from jax.experimental.pallas import tpu as pltpu
from jax.experimental.pallas import tpu_sc as plsc

_SLICES = ((0, 10), (10, 16), (16, 22), (22, 31), (31, 36))
_K = 512
_F = 36
_NC = 5
_COMMIT = 0.25
_BLOCK = 4096

_NUM_WORKERS = 32
_CHUNK = 2048


def _vq_distance_kernel(xt_ref, sf_ref, w_ref, cbn_ref,
                        acc_ref, i0_ref, i1_ref, i2_ref, i3_ref, i4_ref):
    idx_refs = (i0_ref, i1_ref, i2_ref, i3_ref, i4_ref)
    xt = xt_ref[...]
    mm = jnp.dot(w_ref[...], xt, preferred_element_type=jnp.float32)
    loss_lanes = lax.broadcasted_iota(jnp.int32, (1, 128), 1)
    acc_update = jnp.zeros((1, 128), dtype=jnp.float32)
    sub_iota = lax.broadcasted_iota(jnp.int32, (8, mm.shape[1]), 0)

    def comb(av, ai, bv, bi):
        take_b = (bv < av) | ((bv == av) & (bi < ai))
        return jnp.where(take_b, bv, av), jnp.where(take_b, bi, ai)

    for c in range(_NC):
        sf = sf_ref[c:c + 1, :]
        d = sf + cbn_ref[c * _K:(c + 1) * _K, :] + mm[c * _K:(c + 1) * _K, :]
        val = d[0:8, :]
        sid = jnp.zeros(val.shape, jnp.int32)
        for i in range(1, _K // 8):
            b = d[8 * i:8 * i + 8, :]
            cm = b < val
            val = jnp.minimum(val, b)
            sid = jnp.where(cm, i, sid)
        idx8 = sid * 8 + sub_iota
        v4, i4 = comb(val[0:4, :], idx8[0:4, :], val[4:8, :], idx8[4:8, :])
        v2, i2 = comb(v4[0:2, :], i4[0:2, :], v4[2:4, :], i4[2:4, :])
        m, idx = comb(v2[0:1, :], i2[0:1, :], v2[1:2, :], i2[1:2, :])
        idx_refs[c][...] = idx
        acc_update = acc_update + jnp.where(loss_lanes == c, jnp.sum(m), 0.0)

    @pl.when(pl.program_id(0) == 0)
    def _init():
        acc_ref[...] = jnp.zeros_like(acc_ref)

    acc_ref[...] += acc_update


def _make_gather_kernel(B):
    rows_per_worker = B // _NUM_WORKERS
    chunk = _CHUNK
    while rows_per_worker % chunk:
        chunk //= 2
    n_chunks = rows_per_worker // chunk
    n_groups = chunk // 16
    mesh = plsc.VectorSubcoreMesh(core_axis_name="c", subcore_axis_name="s")
    scratch = ([pltpu.VMEM((_K * (e - s),), jnp.float32) for (s, e) in _SLICES]
               + [pltpu.VMEM((chunk,), jnp.int32) for _ in range(_NC)]
               + [pltpu.VMEM((chunk * _F,), jnp.float32),
                  pltpu.SemaphoreType.DMA])

    @functools.partial(
        pl.kernel, mesh=mesh,
        out_type=jax.ShapeDtypeStruct((B * _F,), jnp.float32),
        scratch_types=scratch,
        compiler_params=pltpu.CompilerParams(needs_layout_passes=False),
    )
    def gather_kernel(i0, i1, i2, i3, i4, cb0, cb1, cb2, cb3, cb4, out_hbm,
                      cbv0, cbv1, cbv2, cbv3, cbv4,
                      iv0, iv1, iv2, iv3, iv4, out_v, sem):
        idx_hbm = (i0, i1, i2, i3, i4)
        cb_hbm = (cb0, cb1, cb2, cb3, cb4)
        cb_v = (cbv0, cbv1, cbv2, cbv3, cbv4)
        idx_v = (iv0, iv1, iv2, iv3, iv4)
        wid = lax.axis_index("s") * 2 + lax.axis_index("c")
        base = wid * rows_per_worker
        lane = lax.iota(jnp.int32, 16)

        def chunk_body(ch, _):
            row0 = base + ch * chunk
            handles = [pltpu.async_copy(cb_hbm[c], cb_v[c], sem)
                       for c in range(_NC)]
            handles += [
                pltpu.async_copy(idx_hbm[c].at[pl.ds(row0, chunk)],
                                 idx_v[c], sem)
                for c in range(_NC)]
            for h in handles:
                h.wait()

            def group_body(g, _):
                rows36 = (g * 16 + lane) * _F
                for c, (s, e) in enumerate(_SLICES):
                    fidx = idx_v[c][pl.ds(g * 16, 16)] * (e - s)
                    for dd in range(e - s):
                        vals = plsc.load_gather(cb_v[c], [fidx + dd])
                        plsc.store_scatter(out_v, [rows36 + (s + dd)], vals)
                return 0

            lax.fori_loop(0, n_groups, group_body, 0)
            pltpu.sync_copy(out_v, out_hbm.at[pl.ds(row0 * _F, chunk * _F)])
            return 0

        lax.fori_loop(0, n_chunks, chunk_body, 0)

    return gather_kernel


_PART_FRACS = (1, 1, 1, 1)


@jax.jit
def kernel(features, cb_handshape, cb_location, cb_orientation, cb_movement,
           cb_nonmanual):
    cbs = (cb_handshape, cb_location, cb_orientation, cb_movement, cb_nonmanual)
    B = features.shape[0]

    w = jnp.zeros((_NC * _K, _F), dtype=jnp.float32)
    cbn_parts = []
    for c, (s, e) in enumerate(_SLICES):
        w = w.at[c * _K:(c + 1) * _K, s:e].set(-2.0 * cbs[c])
        cbn_parts.append(jnp.sum(cbs[c] ** 2, axis=1))
    cbn = jnp.concatenate(cbn_parts).reshape(_NC * _K, 1)
    cbs_flat = tuple(cb.reshape(-1) for cb in cbs)

    unit = B // sum(_PART_FRACS)
    part_sizes = tuple(f * unit for f in _PART_FRACS)
    part_starts = tuple(sum(part_sizes[:p]) for p in range(len(part_sizes)))
    idx_spec = pl.BlockSpec((1, _BLOCK), lambda i: (0, i))

    accs, idx_parts, q_parts = [], [], []
    for p, (p0, Bp) in enumerate(zip(part_starts, part_sizes)):
        nblk = Bp // _BLOCK
        out_shapes = (
            jax.ShapeDtypeStruct((1, 128), jnp.float32),
        ) + tuple(jax.ShapeDtypeStruct((1, Bp), jnp.int32) for _ in range(_NC))
        gather_fn = _make_gather_kernel(Bp)
        fp = lax.slice_in_dim(features, p0, p0 + Bp, axis=0)
        sf_all = jnp.concatenate(
            [jnp.sum(fp[:, s:e] ** 2, axis=1)[None, :] for (s, e) in _SLICES]
            + [jnp.zeros((3, Bp), jnp.float32)], axis=0)
        xt = fp.T
        outs = pl.pallas_call(
            _vq_distance_kernel,
            grid=(nblk,),
            in_specs=[
                pl.BlockSpec((_F, _BLOCK), lambda i: (0, i)),
                pl.BlockSpec((8, _BLOCK), lambda i: (0, i)),
                pl.BlockSpec((_NC * _K, _F), lambda i: (0, 0)),
                pl.BlockSpec((_NC * _K, 1), lambda i: (0, 0)),
            ],
            out_specs=(
                pl.BlockSpec((1, 128), lambda i: (0, 0)),
            ) + tuple(idx_spec for _ in range(_NC)),
            out_shape=out_shapes,
        )(xt, sf_all, w, cbn)
        accs.append(outs[0])
        idxs = tuple(o.reshape(Bp) for o in outs[1:])
        idx_parts.append(idxs)
        q_parts.append(gather_fn(*idxs, *cbs_flat))

    acc = sum(accs[1:], accs[0])
    indices = tuple(
        jnp.concatenate([ip[c] for ip in idx_parts]) for c in range(_NC))
    quantized_st = jnp.concatenate(q_parts).reshape(B, _F)

    dims = jnp.array([float(e - s) for (s, e) in _SLICES], dtype=jnp.float32)
    sums = acc[0, :_NC]
    losses = (1.0 + _COMMIT) * sums / (B * dims)
    vq_loss = jnp.mean(losses)
    return (quantized_st, vq_loss) + indices


# speedup vs baseline: 1.0975x; 1.0975x over previous
import functools

import jax
import jax.numpy as jnp
from jax import lax
from jax.experimental import pallas as pl
from jax.experimental.pallas import tpu as pltpu
from jax.experimental.pallas import tpu_sc as plsc

_SLICES = ((0, 10), (10, 16), (16, 22), (22, 31), (31, 36))
_K = 512
_F = 36
_NC = 5
_COMMIT = 0.25
_BLOCK = 4096

_NUM_WORKERS = 32
_CHUNK = 2048


def _vq_distance_kernel(xt_ref, sf_ref, w_ref, cbn_ref,
                        acc_ref, i0_ref, i1_ref, i2_ref, i3_ref, i4_ref):
    idx_refs = (i0_ref, i1_ref, i2_ref, i3_ref, i4_ref)
    xt = xt_ref[...]
    mm = jnp.dot(w_ref[...], xt, preferred_element_type=jnp.float32)
    loss_lanes = lax.broadcasted_iota(jnp.int32, (1, 128), 1)
    acc_update = jnp.zeros((1, 128), dtype=jnp.float32)
    sub_iota = lax.broadcasted_iota(jnp.int32, (8, mm.shape[1]), 0)

    def comb(av, ai, bv, bi):
        take_b = (bv < av) | ((bv == av) & (bi < ai))
        return jnp.where(take_b, bv, av), jnp.where(take_b, bi, ai)

    for c in range(_NC):
        sf = sf_ref[c:c + 1, :]
        d = sf + cbn_ref[c * _K:(c + 1) * _K, :] + mm[c * _K:(c + 1) * _K, :]
        val = d[0:8, :]
        sid = jnp.zeros(val.shape, jnp.int32)
        for i in range(1, _K // 8):
            b = d[8 * i:8 * i + 8, :]
            cm = b < val
            val = jnp.minimum(val, b)
            sid = jnp.where(cm, i, sid)
        idx8 = sid * 8 + sub_iota
        v4, i4 = comb(val[0:4, :], idx8[0:4, :], val[4:8, :], idx8[4:8, :])
        v2, i2 = comb(v4[0:2, :], i4[0:2, :], v4[2:4, :], i4[2:4, :])
        m, idx = comb(v2[0:1, :], i2[0:1, :], v2[1:2, :], i2[1:2, :])
        idx_refs[c][...] = idx
        acc_update = acc_update + jnp.where(loss_lanes == c, jnp.sum(m), 0.0)

    @pl.when(pl.program_id(0) == 0)
    def _init():
        acc_ref[...] = jnp.zeros_like(acc_ref)

    acc_ref[...] += acc_update


def _make_gather_kernel(B):
    rows_per_worker = B // _NUM_WORKERS
    chunk = _CHUNK
    while rows_per_worker % chunk:
        chunk //= 2
    n_chunks = rows_per_worker // chunk
    n_groups = chunk // 16
    mesh = plsc.VectorSubcoreMesh(core_axis_name="c", subcore_axis_name="s")
    scratch = ([pltpu.VMEM((_K * (e - s),), jnp.float32) for (s, e) in _SLICES]
               + [pltpu.VMEM((chunk,), jnp.int32) for _ in range(_NC)]
               + [pltpu.VMEM((chunk * _F,), jnp.float32),
                  pltpu.SemaphoreType.DMA])

    @functools.partial(
        pl.kernel, mesh=mesh,
        out_type=jax.ShapeDtypeStruct((B * _F,), jnp.float32),
        scratch_types=scratch,
        compiler_params=pltpu.CompilerParams(needs_layout_passes=False),
    )
    def gather_kernel(i0, i1, i2, i3, i4, cb0, cb1, cb2, cb3, cb4, out_hbm,
                      cbv0, cbv1, cbv2, cbv3, cbv4,
                      iv0, iv1, iv2, iv3, iv4, out_v, sem):
        idx_hbm = (i0, i1, i2, i3, i4)
        cb_hbm = (cb0, cb1, cb2, cb3, cb4)
        cb_v = (cbv0, cbv1, cbv2, cbv3, cbv4)
        idx_v = (iv0, iv1, iv2, iv3, iv4)
        wid = lax.axis_index("s") * 2 + lax.axis_index("c")
        base = wid * rows_per_worker
        lane = lax.iota(jnp.int32, 16)

        def chunk_body(ch, _):
            row0 = base + ch * chunk
            handles = [pltpu.async_copy(cb_hbm[c], cb_v[c], sem)
                       for c in range(_NC)]
            handles += [
                pltpu.async_copy(idx_hbm[c].at[pl.ds(row0, chunk)],
                                 idx_v[c], sem)
                for c in range(_NC)]
            for h in handles:
                h.wait()

            def group_body(g, _):
                rows36 = (g * 16 + lane) * _F
                for c, (s, e) in enumerate(_SLICES):
                    fidx = idx_v[c][pl.ds(g * 16, 16)] * (e - s)
                    for dd in range(e - s):
                        vals = plsc.load_gather(cb_v[c], [fidx + dd])
                        plsc.store_scatter(out_v, [rows36 + (s + dd)], vals)
                return 0

            lax.fori_loop(0, n_groups, group_body, 0)
            pltpu.sync_copy(out_v, out_hbm.at[pl.ds(row0 * _F, chunk * _F)])
            return 0

        lax.fori_loop(0, n_chunks, chunk_body, 0)

    return gather_kernel


_PART_FRACS = (1, 1)


@jax.jit
def kernel(features, cb_handshape, cb_location, cb_orientation, cb_movement,
           cb_nonmanual):
    cbs = (cb_handshape, cb_location, cb_orientation, cb_movement, cb_nonmanual)
    B = features.shape[0]

    w = jnp.zeros((_NC * _K, _F), dtype=jnp.float32)
    cbn_parts = []
    for c, (s, e) in enumerate(_SLICES):
        w = w.at[c * _K:(c + 1) * _K, s:e].set(-2.0 * cbs[c])
        cbn_parts.append(jnp.sum(cbs[c] ** 2, axis=1))
    cbn = jnp.concatenate(cbn_parts).reshape(_NC * _K, 1)
    cbs_flat = tuple(cb.reshape(-1) for cb in cbs)

    unit = B // sum(_PART_FRACS)
    part_sizes = tuple(f * unit for f in _PART_FRACS)
    part_starts = tuple(sum(part_sizes[:p]) for p in range(len(part_sizes)))
    idx_spec = pl.BlockSpec((1, _BLOCK), lambda i: (0, i))

    accs, idx_parts, q_parts = [], [], []
    for p, (p0, Bp) in enumerate(zip(part_starts, part_sizes)):
        nblk = Bp // _BLOCK
        out_shapes = (
            jax.ShapeDtypeStruct((1, 128), jnp.float32),
        ) + tuple(jax.ShapeDtypeStruct((1, Bp), jnp.int32) for _ in range(_NC))
        gather_fn = _make_gather_kernel(Bp)
        fp = lax.slice_in_dim(features, p0, p0 + Bp, axis=0)
        sf_all = jnp.concatenate(
            [jnp.sum(fp[:, s:e] ** 2, axis=1)[None, :] for (s, e) in _SLICES]
            + [jnp.zeros((3, Bp), jnp.float32)], axis=0)
        xt = fp.T
        outs = pl.pallas_call(
            _vq_distance_kernel,
            grid=(nblk,),
            in_specs=[
                pl.BlockSpec((_F, _BLOCK), lambda i: (0, i)),
                pl.BlockSpec((8, _BLOCK), lambda i: (0, i)),
                pl.BlockSpec((_NC * _K, _F), lambda i: (0, 0)),
                pl.BlockSpec((_NC * _K, 1), lambda i: (0, 0)),
            ],
            out_specs=(
                pl.BlockSpec((1, 128), lambda i: (0, 0)),
            ) + tuple(idx_spec for _ in range(_NC)),
            out_shape=out_shapes,
        )(xt, sf_all, w, cbn)
        accs.append(outs[0])
        idxs = tuple(o.reshape(Bp) for o in outs[1:])
        idx_parts.append(idxs)
        q_parts.append(gather_fn(*idxs, *cbs_flat))

    acc = sum(accs[1:], accs[0])
    indices = tuple(
        jnp.concatenate([ip[c] for ip in idx_parts]) for c in range(_NC))
    quantized_st = jnp.concatenate(q_parts).reshape(B, _F)

    dims = jnp.array([float(e - s) for (s, e) in _SLICES], dtype=jnp.float32)
    sums = acc[0, :_NC]
    losses = (1.0 + _COMMIT) * sums / (B * dims)
    vq_loss = jnp.mean(losses)
    return (quantized_st, vq_loss) + indices
